# Initial kernel scaffold; baseline (speedup 1.0000x reference)
import jax
import jax.numpy as jnp
from jax.experimental import pallas as pl


def kernel(x, pos_emb):
    raise NotImplementedError("write your pallas kernel here")


# speedup vs baseline: 1.7232x; 1.7232x over previous
import jax
import jax.numpy as jnp
from jax.experimental import pallas as pl

BT = 512


def _add_pe_kernel(x_ref, pe_ref, o_ref):
    o_ref[...] = x_ref[...] + pe_ref[...][None, :, :]


def kernel(x, pos_emb):
    B, T, C = x.shape
    pe = pos_emb[:T]
    grid = (T // BT,)
    return pl.pallas_call(
        _add_pe_kernel,
        grid=grid,
        in_specs=[
            pl.BlockSpec((B, BT, C), lambda t: (0, t, 0)),
            pl.BlockSpec((BT, C), lambda t: (t, 0)),
        ],
        out_specs=pl.BlockSpec((B, BT, C), lambda t: (0, t, 0)),
        out_shape=jax.ShapeDtypeStruct((B, T, C), x.dtype),
    )(x, pe)
